# Initial kernel scaffold; baseline (speedup 1.0000x reference)
import jax
import jax.numpy as jnp
from jax.experimental import pallas as pl


def kernel(x, edge_index, W1, att_src1, att_dst1, b1, W2, att_src2, att_dst2, b2):
    raise NotImplementedError("write your pallas kernel here")


# speedup vs baseline: 12.9306x; 12.9306x over previous
import dataclasses
import functools

import jax
import jax.numpy as jnp
from jax import lax
from jax.experimental import pallas as pl
from jax.experimental.pallas import tpu as pltpu
from jax.experimental.pallas import tpu_sc as plsc

N = 10000
D = 128
E = 320000
B = 64
NBLK = E // B
ACCW = 144
NC = 2
NS = 16
NW = NC * NS
KMAX = (NBLK + NW - 1) // NW
ROWS_PER_TILE = N // NS

_HIGH = lax.Precision.HIGHEST


def _lrelu(z):
    return jnp.maximum(z, z * 0.2)


def _pre_body(x_ref, w_ref, asv_ref, adv_ref, h_ref, asrc_ref, adst_ref,
              gmax_ref, exl_ref):
    h = jnp.dot(x_ref[...], w_ref[...], preferred_element_type=jnp.float32,
                precision=_HIGH)
    h_ref[...] = h
    a_src = jnp.sum(h * asv_ref[...], axis=1, keepdims=True)
    a_dst = jnp.sum(h * adv_ref[...], axis=1, keepdims=True)
    gmax = jnp.max(a_src)
    m = _lrelu(a_dst + gmax)
    asrc_ref[...] = a_src
    adst_ref[...] = a_dst
    gmax_ref[...] = jnp.broadcast_to(gmax, (1, 1))
    exl_ref[...] = jnp.exp(_lrelu(a_src + a_dst) - m)


_pre = pl.pallas_call(
    _pre_body,
    out_shape=[
        jax.ShapeDtypeStruct((N, D), jnp.float32),
        jax.ShapeDtypeStruct((N, 1), jnp.float32),
        jax.ShapeDtypeStruct((N, 1), jnp.float32),
        jax.ShapeDtypeStruct((1, 1), jnp.float32),
        jax.ShapeDtypeStruct((N, 1), jnp.float32),
    ],
)


def _post_body(acc_ref, h_ref, exl_ref, b_ref, o_ref, *, act):
    s = acc_ref[0] + acc_ref[1]
    exl = exl_ref[...]
    num = s[:, :D] + exl * h_ref[...]
    den = s[:, D:D + 1] + exl + 1e-16
    out = num / den + b_ref[...]
    if act:
        out = jnp.where(out > 0, out, jnp.exp(out) - 1.0)
    o_ref[...] = out


def _make_post(act):
    return pl.pallas_call(
        functools.partial(_post_body, act=act),
        out_shape=jax.ShapeDtypeStruct((N, D), jnp.float32),
    )


_post_elu = _make_post(True)
_post_lin = _make_post(False)


_cp = pltpu.CompilerParams()
if "needs_layout_passes" in pltpu.CompilerParams.__dataclass_fields__:
    _cp = dataclasses.replace(_cp, needs_layout_passes=False)
if "use_tc_tiling_on_sc" in pltpu.CompilerParams.__dataclass_fields__:
    _cp = dataclasses.replace(_cp, use_tc_tiling_on_sc=False)

_mesh = plsc.VectorSubcoreMesh(core_axis_name="c", subcore_axis_name="s")


@functools.partial(
    pl.kernel,
    out_type=jax.ShapeDtypeStruct((NC, N, ACCW), jnp.float32),
    mesh=_mesh,
    scratch_types=[
        pltpu.VMEM((N,), jnp.float32),
        pltpu.VMEM((N,), jnp.float32),
        pltpu.VMEM((16,), jnp.float32),
        pltpu.VMEM((B,), jnp.int32),
        pltpu.VMEM((B,), jnp.int32),
        pltpu.VMEM((B,), jnp.float32),
        pltpu.VMEM((B, D), jnp.float32),
        pltpu.VMEM((B, ACCW), jnp.float32),
        pltpu.VMEM_SHARED((N, ACCW), jnp.float32),
    ],
    compiler_params=_cp,
)
def _sc_gat(ei_hbm, h_hbm, asrc_hbm, adst_hbm, gmax_hbm, out_hbm,
            asrc_t, adst_t, gmax_t, src_v, dst_v, ex_v, raw_v, scaled_v,
            acc_sh):
    cid = lax.axis_index("c")
    sid = lax.axis_index("s")
    wid = sid * NC + cid
    lane0 = (lax.iota(jnp.int32, 16) == 0).astype(jnp.float32)
    zeros16 = jnp.zeros((16,), jnp.float32)
    izeros16 = jnp.zeros((16,), jnp.int32)

    pltpu.sync_copy(asrc_hbm, asrc_t)
    pltpu.sync_copy(adst_hbm, adst_t)
    pltpu.sync_copy(gmax_hbm, gmax_t)
    gmax = gmax_t[...]

    @pl.loop(0, B)
    def _(r):
        @pl.loop(0, ACCW, step=16)
        def _(c):
            scaled_v[r, pl.ds(c, 16)] = zeros16

    @pl.loop(0, ROWS_PER_TILE // B)
    def _(i):
        pltpu.sync_copy(
            scaled_v, acc_sh.at[pl.ds(sid * ROWS_PER_TILE + i * B, B)])

    pltpu.sync_copy(
        scaled_v.at[pl.ds(0, ROWS_PER_TILE % B)],
        acc_sh.at[pl.ds(sid * ROWS_PER_TILE
                        + (ROWS_PER_TILE // B) * B, ROWS_PER_TILE % B)])

    plsc.subcore_barrier()

    @pl.loop(0, KMAX)
    def _(k):
        b = wid + k * NW

        @pl.when(b < NBLK)
        def _():
            off = b * B
            pltpu.sync_copy(ei_hbm.at[0, pl.ds(off, B)], src_v)
            pltpu.sync_copy(ei_hbm.at[1, pl.ds(off, B)], dst_v)
            pltpu.sync_copy(h_hbm.at[src_v], raw_v)

            @pl.loop(0, B, step=16)
            def _(g):
                s16 = src_v[pl.ds(g, 16)]
                d16 = dst_v[pl.ds(g, 16)]
                a_s = plsc.load_gather(asrc_t, [s16])
                a_d = plsc.load_gather(adst_t, [d16])
                m_d = _lrelu(a_d + gmax)
                ex_v[pl.ds(g, 16)] = jnp.exp(_lrelu(a_s + a_d) - m_d)

            @pl.loop(0, B)
            def _(r):
                exr = plsc.load_gather(ex_v, [izeros16 + r])
                for c in range(D // 16):
                    scaled_v[r, pl.ds(c * 16, 16)] = (
                        raw_v[r, pl.ds(c * 16, 16)] * exr)
                scaled_v[r, pl.ds(D, 16)] = exr * lane0

            pltpu.sync_copy(scaled_v, acc_sh.at[dst_v], add=True)

    plsc.subcore_barrier()
    pltpu.sync_copy(
        acc_sh.at[pl.ds(sid * ROWS_PER_TILE, ROWS_PER_TILE)],
        out_hbm.at[cid, pl.ds(sid * ROWS_PER_TILE, ROWS_PER_TILE)])


def _gat_layer(x, edge_index, W, att_src, att_dst, bias, act):
    asv = att_src.reshape(1, D).astype(jnp.float32)
    adv = att_dst.reshape(1, D).astype(jnp.float32)
    h, a_src, a_dst, gmax, exl = _pre(x, W, asv, adv)
    acc = _sc_gat(edge_index, h, a_src.reshape(N), a_dst.reshape(N),
                  jnp.broadcast_to(gmax.reshape(()), (16,)))
    post = _post_elu if act else _post_lin
    return post(acc, h, exl, bias.reshape(1, D))


def kernel(x, edge_index, W1, att_src1, att_dst1, b1, W2, att_src2,
           att_dst2, b2):
    h1 = _gat_layer(x, edge_index, W1, att_src1, att_dst1, b1, act=True)
    return _gat_layer(h1, edge_index, W2, att_src2, att_dst2, b2, act=False)


# speedup vs baseline: 25.0929x; 1.9406x over previous
import dataclasses
import functools

import jax
import jax.numpy as jnp
from jax import lax
from jax.experimental import pallas as pl
from jax.experimental.pallas import tpu as pltpu
from jax.experimental.pallas import tpu_sc as plsc

N = 10000
D = 128
E = 320000
B = 64
NBLK = E // B
ACCW = 144
NC = 2
NS = 16
NW = NC * NS
KMAX = (NBLK + NW - 1) // NW
ROWS_PER_TILE = N // NS

_HIGH = lax.Precision.HIGHEST


def _lrelu(z):
    return jnp.maximum(z, z * 0.2)


def _pre_body(x_ref, w_ref, asv_ref, adv_ref, h_ref, asrc_ref, adst_ref,
              gmax_ref, exl_ref):
    h = jnp.dot(x_ref[...], w_ref[...], preferred_element_type=jnp.float32,
                precision=_HIGH)
    h_ref[...] = h
    a_src = jnp.sum(h * asv_ref[...], axis=1, keepdims=True)
    a_dst = jnp.sum(h * adv_ref[...], axis=1, keepdims=True)
    gmax = jnp.max(a_src)
    m = _lrelu(a_dst + gmax)
    asrc_ref[...] = a_src
    adst_ref[...] = a_dst
    gmax_ref[...] = jnp.broadcast_to(gmax, (1, 1))
    exl_ref[...] = jnp.exp(_lrelu(a_src + a_dst) - m)


_pre = pl.pallas_call(
    _pre_body,
    out_shape=[
        jax.ShapeDtypeStruct((N, D), jnp.float32),
        jax.ShapeDtypeStruct((N, 1), jnp.float32),
        jax.ShapeDtypeStruct((N, 1), jnp.float32),
        jax.ShapeDtypeStruct((1, 1), jnp.float32),
        jax.ShapeDtypeStruct((N, 1), jnp.float32),
    ],
)


def _post_body(acc_ref, h_ref, exl_ref, b_ref, o_ref, *, act):
    s = acc_ref[0] + acc_ref[1]
    exl = exl_ref[...]
    num = s[:, :D] + exl * h_ref[...]
    den = s[:, D:D + 1] + exl + 1e-16
    out = num / den + b_ref[...]
    if act:
        out = jnp.where(out > 0, out, jnp.exp(out) - 1.0)
    o_ref[...] = out


def _make_post(act):
    return pl.pallas_call(
        functools.partial(_post_body, act=act),
        out_shape=jax.ShapeDtypeStruct((N, D), jnp.float32),
    )


_post_elu = _make_post(True)
_post_lin = _make_post(False)


_cp = pltpu.CompilerParams()
if "needs_layout_passes" in pltpu.CompilerParams.__dataclass_fields__:
    _cp = dataclasses.replace(_cp, needs_layout_passes=False)
if "use_tc_tiling_on_sc" in pltpu.CompilerParams.__dataclass_fields__:
    _cp = dataclasses.replace(_cp, use_tc_tiling_on_sc=False)

_mesh = plsc.VectorSubcoreMesh(core_axis_name="c", subcore_axis_name="s")


@functools.partial(
    pl.kernel,
    out_type=jax.ShapeDtypeStruct((NC, N, ACCW), jnp.float32),
    mesh=_mesh,
    scratch_types=[
        pltpu.VMEM((N,), jnp.float32),
        pltpu.VMEM((N,), jnp.float32),
        pltpu.VMEM((16,), jnp.float32),
        pltpu.VMEM((2, B), jnp.int32),
        pltpu.VMEM((2, B), jnp.int32),
        pltpu.VMEM((B,), jnp.float32),
        pltpu.VMEM((B, D // 2), jnp.int32),
        pltpu.VMEM((B, D // 2), jnp.int32),
        pltpu.VMEM((B,), jnp.int32),
        pltpu.VMEM((B, ACCW), jnp.float32),
        pltpu.VMEM_SHARED((N, ACCW), jnp.float32),
        pltpu.SemaphoreType.DMA,
        pltpu.SemaphoreType.DMA,
        pltpu.SemaphoreType.DMA,
    ],
    compiler_params=_cp,
)
def _sc_gat(ei_hbm, hbi_hbm, asrc_hbm, adst_hbm, gmax_hbm, out_hbm,
            asrc_t, adst_t, gmax_t, src_v, dst_v, ex_v, raw0, raw1,
            dst_sc, scaled_v, acc_sh, si, sr, ss):
    cid = lax.axis_index("c")
    sid = lax.axis_index("s")
    wid = sid * NC + cid
    lane0 = (lax.iota(jnp.int32, 16) == 0).astype(jnp.float32)
    zeros16 = jnp.zeros((16,), jnp.float32)
    izeros16 = jnp.zeros((16,), jnp.int32)
    himask = jnp.full((16,), -65536, jnp.int32)

    pltpu.sync_copy(asrc_hbm, asrc_t)
    pltpu.sync_copy(adst_hbm, adst_t)
    pltpu.sync_copy(gmax_hbm, gmax_t)
    gmax = gmax_t[...]

    @pl.loop(0, B)
    def _(r):
        @pl.loop(0, ACCW, step=16)
        def _(c):
            scaled_v[r, pl.ds(c, 16)] = zeros16

    @pl.loop(0, ROWS_PER_TILE // B)
    def _(i):
        pltpu.sync_copy(
            scaled_v, acc_sh.at[pl.ds(sid * ROWS_PER_TILE + i * B, B)])

    pltpu.sync_copy(
        scaled_v.at[pl.ds(0, ROWS_PER_TILE % B)],
        acc_sh.at[pl.ds(sid * ROWS_PER_TILE
                        + (ROWS_PER_TILE // B) * B, ROWS_PER_TILE % B)])

    plsc.subcore_barrier()

    raws = (raw0, raw1)

    def valid(k):
        return wid + k * NW < NBLK

    def idx_start(k, q):
        off = (wid + k * NW) * B
        c0 = pltpu.make_async_copy(ei_hbm.at[0, pl.ds(off, B)],
                                   src_v.at[q], si)
        c1 = pltpu.make_async_copy(ei_hbm.at[1, pl.ds(off, B)],
                                   dst_v.at[q], si)
        c0.start()
        c1.start()
        return (c0, c1)

    idx_start(0, 0)
    pltpu.make_async_copy(ei_hbm.at[0, pl.ds(wid * B, B)],
                          src_v.at[0], si).wait()
    pltpu.make_async_copy(ei_hbm.at[1, pl.ds(wid * B, B)],
                          dst_v.at[0], si).wait()
    g0 = pltpu.make_async_copy(hbi_hbm.at[src_v.at[0]], raw0, sr)
    g0.start()
    g0.wait()
    idx_start(1, 1)
    pltpu.make_async_copy(ei_hbm.at[0, pl.ds((wid + NW) * B, B)],
                          src_v.at[1], si).wait()
    pltpu.make_async_copy(ei_hbm.at[1, pl.ds((wid + NW) * B, B)],
                          dst_v.at[1], si).wait()

    def body(k0, k, q):
        kk = k0 + k
        p = q

        @pl.when(valid(kk))
        def _():
            qn = 1 - q

            @pl.when(valid(kk + 1))
            def _():
                pltpu.make_async_copy(
                    hbi_hbm.at[src_v.at[qn]], raws[1 - p], sr).start()

            @pl.loop(0, B, step=16)
            def _(g):
                s16 = src_v[q, pl.ds(g, 16)]
                d16 = dst_v[q, pl.ds(g, 16)]
                a_s = plsc.load_gather(asrc_t, [s16])
                a_d = plsc.load_gather(adst_t, [d16])
                m_d = _lrelu(a_d + gmax)
                ex_v[pl.ds(g, 16)] = jnp.exp(_lrelu(a_s + a_d) - m_d)

            raw = raws[p]

            @pl.loop(0, B)
            def _(r):
                exr = plsc.load_gather(ex_v, [izeros16 + r])
                for c in range(D // 32):
                    w = raw[r, pl.ds(c * 16, 16)]
                    lo = plsc.bitcast(w << 16, jnp.float32)
                    hi = plsc.bitcast(w & himask, jnp.float32)
                    scaled_v[r, pl.ds(c * 32, 16)] = lo * exr
                    scaled_v[r, pl.ds(c * 32 + 16, 16)] = hi * exr
                scaled_v[r, pl.ds(D, 16)] = exr * lane0

            @pl.loop(0, B, step=16)
            def _(g):
                dst_sc[pl.ds(g, 16)] = dst_v[q, pl.ds(g, 16)]

            sc = pltpu.async_copy(scaled_v, acc_sh.at[dst_sc], ss, add=True)

            @pl.when(valid(kk + 2))
            def _():
                off2 = (wid + (kk + 2) * NW) * B
                c0 = pltpu.make_async_copy(ei_hbm.at[0, pl.ds(off2, B)],
                                           src_v.at[q], si)
                c1 = pltpu.make_async_copy(ei_hbm.at[1, pl.ds(off2, B)],
                                           dst_v.at[q], si)
                c0.start()
                c1.start()
                c0.wait()
                c1.wait()

            sc.wait()

            @pl.when(valid(kk + 1))
            def _():
                pltpu.make_async_copy(
                    hbi_hbm.at[src_v.at[qn]], raws[1 - p], sr).wait()

    @pl.loop(0, KMAX + (-KMAX) % 2, step=2)
    def _(k0):
        body(k0, 0, 0)
        body(k0, 1, 1)

    plsc.subcore_barrier()
    pltpu.sync_copy(
        acc_sh.at[pl.ds(sid * ROWS_PER_TILE, ROWS_PER_TILE)],
        out_hbm.at[cid, pl.ds(sid * ROWS_PER_TILE, ROWS_PER_TILE)])


def _gat_layer(x, edge_index, W, att_src, att_dst, bias, act):
    asv = att_src.reshape(1, D).astype(jnp.float32)
    adv = att_dst.reshape(1, D).astype(jnp.float32)
    h, a_src, a_dst, gmax, exl = _pre(x, W, asv, adv)
    hb = h.reshape(N, 4, 2, 16).transpose(0, 1, 3, 2).astype(
        jnp.bfloat16).reshape(N, D // 2, 2)
    hbi = jax.lax.bitcast_convert_type(hb, jnp.int32)
    acc = _sc_gat(edge_index, hbi, a_src.reshape(N), a_dst.reshape(N),
                  jnp.broadcast_to(gmax.reshape(()), (16,)))
    post = _post_elu if act else _post_lin
    return post(acc, h, exl, bias.reshape(1, D))


def kernel(x, edge_index, W1, att_src1, att_dst1, b1, W2, att_src2,
           att_dst2, b2):
    h1 = _gat_layer(x, edge_index, W1, att_src1, att_dst1, b1, act=True)
    return _gat_layer(h1, edge_index, W2, att_src2, att_dst2, b2, act=False)


# speedup vs baseline: 26.5347x; 1.0575x over previous
import dataclasses
import functools

import jax
import jax.numpy as jnp
from jax import lax
from jax.experimental import pallas as pl
from jax.experimental.pallas import tpu as pltpu
from jax.experimental.pallas import tpu_sc as plsc

N = 10000
D = 128
E = 320000
B = 128
NBLK = E // B
ACCW = 144
NC = 2
NS = 16
NW = NC * NS
KMAX = (NBLK + NW - 1) // NW
ROWS_PER_TILE = N // NS

_HIGH = lax.Precision.HIGHEST


def _lrelu(z):
    return jnp.maximum(z, z * 0.2)


def _pre_body(x_ref, w_ref, asv_ref, adv_ref, h_ref, asrc_ref, adst_ref,
              gmax_ref, exl_ref):
    h = jnp.dot(x_ref[...], w_ref[...], preferred_element_type=jnp.float32,
                precision=_HIGH)
    h_ref[...] = h
    a_src = jnp.sum(h * asv_ref[...], axis=1, keepdims=True)
    a_dst = jnp.sum(h * adv_ref[...], axis=1, keepdims=True)
    gmax = jnp.max(a_src)
    m = _lrelu(a_dst + gmax)
    asrc_ref[...] = a_src
    adst_ref[...] = a_dst
    gmax_ref[...] = jnp.broadcast_to(gmax, (1, 1))
    exl_ref[...] = jnp.exp(_lrelu(a_src + a_dst) - m)


_pre = pl.pallas_call(
    _pre_body,
    out_shape=[
        jax.ShapeDtypeStruct((N, D), jnp.float32),
        jax.ShapeDtypeStruct((N, 1), jnp.float32),
        jax.ShapeDtypeStruct((N, 1), jnp.float32),
        jax.ShapeDtypeStruct((1, 1), jnp.float32),
        jax.ShapeDtypeStruct((N, 1), jnp.float32),
    ],
)


def _post_body(acc_ref, h_ref, exl_ref, b_ref, o_ref, *, act):
    s = acc_ref[0] + acc_ref[1]
    exl = exl_ref[...]
    num = s[:, :D] + exl * h_ref[...]
    den = s[:, D:D + 1] + exl + 1e-16
    out = num / den + b_ref[...]
    if act:
        out = jnp.where(out > 0, out, jnp.exp(out) - 1.0)
    o_ref[...] = out


def _make_post(act):
    return pl.pallas_call(
        functools.partial(_post_body, act=act),
        out_shape=jax.ShapeDtypeStruct((N, D), jnp.float32),
    )


_post_elu = _make_post(True)
_post_lin = _make_post(False)


_cp = pltpu.CompilerParams()
if "needs_layout_passes" in pltpu.CompilerParams.__dataclass_fields__:
    _cp = dataclasses.replace(_cp, needs_layout_passes=False)
if "use_tc_tiling_on_sc" in pltpu.CompilerParams.__dataclass_fields__:
    _cp = dataclasses.replace(_cp, use_tc_tiling_on_sc=False)

_mesh = plsc.VectorSubcoreMesh(core_axis_name="c", subcore_axis_name="s")


@functools.partial(
    pl.kernel,
    out_type=jax.ShapeDtypeStruct((NC, N, ACCW), jnp.float32),
    mesh=_mesh,
    scratch_types=[
        pltpu.VMEM((16,), jnp.float32),
        pltpu.VMEM((2, B), jnp.int32),
        pltpu.VMEM((2, B), jnp.int32),
        pltpu.VMEM((2, B), jnp.float32),
        pltpu.VMEM((2, B), jnp.float32),
        pltpu.VMEM((B,), jnp.float32),
        pltpu.VMEM((B, D // 2), jnp.int32),
        pltpu.VMEM((B, D // 2), jnp.int32),
        pltpu.VMEM((B,), jnp.int32),
        pltpu.VMEM((B, ACCW), jnp.float32),
        pltpu.VMEM_SHARED((N, ACCW), jnp.float32),
        pltpu.SemaphoreType.DMA,
        pltpu.SemaphoreType.DMA,
        pltpu.SemaphoreType.DMA,
    ],
    compiler_params=_cp,
)
def _sc_gat(ei_hbm, hbi_hbm, asrc_hbm, adst_hbm, gmax_hbm, out_hbm,
            gmax_t, src_v, dst_v, av_s, av_d, ex_v, raw0, raw1,
            dst_sc, scaled_v, acc_sh, si, sr, ss):
    cid = lax.axis_index("c")
    sid = lax.axis_index("s")
    wid = sid * NC + cid
    lane0 = (lax.iota(jnp.int32, 16) == 0).astype(jnp.float32)
    zeros16 = jnp.zeros((16,), jnp.float32)
    izeros16 = jnp.zeros((16,), jnp.int32)
    himask = jnp.full((16,), -65536, jnp.int32)

    pltpu.sync_copy(gmax_hbm, gmax_t)
    gmax = gmax_t[...]

    @pl.loop(0, B)
    def _(r):
        @pl.loop(0, ACCW, step=16)
        def _(c):
            scaled_v[r, pl.ds(c, 16)] = zeros16

    @pl.loop(0, ROWS_PER_TILE // B)
    def _(i):
        pltpu.sync_copy(
            scaled_v, acc_sh.at[pl.ds(sid * ROWS_PER_TILE + i * B, B)])

    pltpu.sync_copy(
        scaled_v.at[pl.ds(0, ROWS_PER_TILE % B)],
        acc_sh.at[pl.ds(sid * ROWS_PER_TILE
                        + (ROWS_PER_TILE // B) * B, ROWS_PER_TILE % B)])

    plsc.subcore_barrier()

    raws = (raw0, raw1)

    def valid(k):
        return wid + k * NW < NBLK

    def idx_start(k, q):
        off = (wid + k * NW) * B
        c0 = pltpu.make_async_copy(ei_hbm.at[0, pl.ds(off, B)],
                                   src_v.at[q], si)
        c1 = pltpu.make_async_copy(ei_hbm.at[1, pl.ds(off, B)],
                                   dst_v.at[q], si)
        c0.start()
        c1.start()
        return (c0, c1)

    def edge_gathers(q, p):
        return (pltpu.make_async_copy(hbi_hbm.at[src_v.at[q]], raws[p], sr),
                pltpu.make_async_copy(asrc_hbm.at[src_v.at[q]],
                                      av_s.at[q], sr),
                pltpu.make_async_copy(adst_hbm.at[dst_v.at[q]],
                                      av_d.at[q], sr))

    idx_start(0, 0)
    pltpu.make_async_copy(ei_hbm.at[0, pl.ds(wid * B, B)],
                          src_v.at[0], si).wait()
    pltpu.make_async_copy(ei_hbm.at[1, pl.ds(wid * B, B)],
                          dst_v.at[0], si).wait()
    for g0 in edge_gathers(0, 0):
        g0.start()
        g0.wait()
    idx_start(1, 1)
    pltpu.make_async_copy(ei_hbm.at[0, pl.ds((wid + NW) * B, B)],
                          src_v.at[1], si).wait()
    pltpu.make_async_copy(ei_hbm.at[1, pl.ds((wid + NW) * B, B)],
                          dst_v.at[1], si).wait()

    def body(k0, k, q):
        kk = k0 + k
        p = q

        @pl.when(valid(kk))
        def _():
            qn = 1 - q

            @pl.when(valid(kk + 1))
            def _():
                for g in edge_gathers(qn, 1 - p):
                    g.start()

            @pl.loop(0, B, step=16)
            def _(g):
                a_s = av_s[q, pl.ds(g, 16)]
                a_d = av_d[q, pl.ds(g, 16)]
                m_d = _lrelu(a_d + gmax)
                ex_v[pl.ds(g, 16)] = jnp.exp(_lrelu(a_s + a_d) - m_d)

            raw = raws[p]

            @pl.loop(0, B)
            def _(r):
                exr = plsc.load_gather(ex_v, [izeros16 + r])
                for c in range(D // 32):
                    w = raw[r, pl.ds(c * 16, 16)]
                    lo = plsc.bitcast(w << 16, jnp.float32)
                    hi = plsc.bitcast(w & himask, jnp.float32)
                    scaled_v[r, pl.ds(c * 32, 16)] = lo * exr
                    scaled_v[r, pl.ds(c * 32 + 16, 16)] = hi * exr
                scaled_v[r, pl.ds(D, 16)] = exr * lane0

            @pl.loop(0, B, step=16)
            def _(g):
                dst_sc[pl.ds(g, 16)] = dst_v[q, pl.ds(g, 16)]

            sc = pltpu.async_copy(scaled_v, acc_sh.at[dst_sc], ss, add=True)

            @pl.when(valid(kk + 2))
            def _():
                off2 = (wid + (kk + 2) * NW) * B
                c0 = pltpu.make_async_copy(ei_hbm.at[0, pl.ds(off2, B)],
                                           src_v.at[q], si)
                c1 = pltpu.make_async_copy(ei_hbm.at[1, pl.ds(off2, B)],
                                           dst_v.at[q], si)
                c0.start()
                c1.start()
                c0.wait()
                c1.wait()

            sc.wait()

            @pl.when(valid(kk + 1))
            def _():
                for g in edge_gathers(qn, 1 - p):
                    g.wait()

    @pl.loop(0, KMAX + (-KMAX) % 2, step=2)
    def _(k0):
        body(k0, 0, 0)
        body(k0, 1, 1)

    plsc.subcore_barrier()
    pltpu.sync_copy(
        acc_sh.at[pl.ds(sid * ROWS_PER_TILE, ROWS_PER_TILE)],
        out_hbm.at[cid, pl.ds(sid * ROWS_PER_TILE, ROWS_PER_TILE)])


def _gat_layer(x, edge_index, W, att_src, att_dst, bias, act):
    asv = att_src.reshape(1, D).astype(jnp.float32)
    adv = att_dst.reshape(1, D).astype(jnp.float32)
    h, a_src, a_dst, gmax, exl = _pre(x, W, asv, adv)
    hb = h.reshape(N, 4, 2, 16).transpose(0, 1, 3, 2).astype(
        jnp.bfloat16).reshape(N, D // 2, 2)
    hbi = jax.lax.bitcast_convert_type(hb, jnp.int32)
    acc = _sc_gat(edge_index, hbi, a_src.reshape(N), a_dst.reshape(N),
                  jnp.broadcast_to(gmax.reshape(()), (16,)))
    post = _post_elu if act else _post_lin
    return post(acc, h, exl, bias.reshape(1, D))


def kernel(x, edge_index, W1, att_src1, att_dst1, b1, W2, att_src2,
           att_dst2, b2):
    h1 = _gat_layer(x, edge_index, W1, att_src1, att_dst1, b1, act=True)
    return _gat_layer(h1, edge_index, W2, att_src2, att_dst2, b2, act=False)


# speedup vs baseline: 47.7278x; 1.7987x over previous
import dataclasses
import functools

import jax
import jax.numpy as jnp
from jax import lax
from jax.experimental import pallas as pl
from jax.experimental.pallas import tpu as pltpu
from jax.experimental.pallas import tpu_sc as plsc

N = 10000
D = 128
E = 320000
B = 128
NBLK = E // B
ACCW = 144
NC = 2
NS = 16
NW = NC * NS
KMAX = (NBLK + NW - 1) // NW
ROWS_PER_TILE = N // NS

_HIGH = lax.Precision.HIGHEST


def _lrelu(z):
    return jnp.maximum(z, z * 0.2)


def _pre_body(x_ref, w_ref, asv_ref, adv_ref, h_ref, asrc_ref, adst_ref,
              gmax_ref, exl_ref):
    h = jnp.dot(x_ref[...], w_ref[...], preferred_element_type=jnp.float32,
                precision=_HIGH)
    h_ref[...] = h
    a_src = jnp.sum(h * asv_ref[...], axis=1, keepdims=True)
    a_dst = jnp.sum(h * adv_ref[...], axis=1, keepdims=True)
    gmax = jnp.max(a_src)
    m = _lrelu(a_dst + gmax)
    asrc_ref[...] = a_src
    adst_ref[...] = a_dst
    gmax_ref[...] = jnp.broadcast_to(gmax, (1, 1))
    exl_ref[...] = jnp.exp(_lrelu(a_src + a_dst) - m)


_pre = pl.pallas_call(
    _pre_body,
    out_shape=[
        jax.ShapeDtypeStruct((N, D), jnp.float32),
        jax.ShapeDtypeStruct((N, 1), jnp.float32),
        jax.ShapeDtypeStruct((N, 1), jnp.float32),
        jax.ShapeDtypeStruct((1, 1), jnp.float32),
        jax.ShapeDtypeStruct((N, 1), jnp.float32),
    ],
)


def _post_body(acc_ref, h_ref, exl_ref, b_ref, o_ref, *, act):
    s = acc_ref[0] + acc_ref[1]
    exl = exl_ref[...]
    num = s[:, :D] + exl * h_ref[...]
    den = s[:, D:D + 1] + exl + 1e-16
    out = num / den + b_ref[...]
    if act:
        out = jnp.where(out > 0, out, jnp.exp(out) - 1.0)
    o_ref[...] = out


def _make_post(act):
    return pl.pallas_call(
        functools.partial(_post_body, act=act),
        out_shape=jax.ShapeDtypeStruct((N, D), jnp.float32),
    )


_post_elu = _make_post(True)
_post_lin = _make_post(False)


_cp = pltpu.CompilerParams()
if "needs_layout_passes" in pltpu.CompilerParams.__dataclass_fields__:
    _cp = dataclasses.replace(_cp, needs_layout_passes=False)
if "use_tc_tiling_on_sc" in pltpu.CompilerParams.__dataclass_fields__:
    _cp = dataclasses.replace(_cp, use_tc_tiling_on_sc=False)

_mesh = plsc.VectorSubcoreMesh(core_axis_name="c", subcore_axis_name="s")


@functools.partial(
    pl.kernel,
    out_type=jax.ShapeDtypeStruct((NC, N, ACCW), jnp.float32),
    mesh=_mesh,
    scratch_types=[
        pltpu.VMEM((16,), jnp.float32),
        pltpu.VMEM((2, B), jnp.int32),
        pltpu.VMEM((2, B), jnp.int32),
        pltpu.VMEM((2, B), jnp.float32),
        pltpu.VMEM((2, B), jnp.float32),
        pltpu.VMEM((B,), jnp.float32),
        pltpu.VMEM((B, D // 2), jnp.int32),
        pltpu.VMEM((B, D // 2), jnp.int32),
        pltpu.VMEM((B,), jnp.int32),
        pltpu.VMEM((B, ACCW), jnp.float32),
        pltpu.VMEM_SHARED((N, ACCW), jnp.float32),
        pltpu.SemaphoreType.DMA,
        pltpu.SemaphoreType.DMA,
        pltpu.SemaphoreType.DMA,
    ],
    compiler_params=_cp,
)
def _sc_gat(ei_hbm, hbi_hbm, asrc_hbm, adst_hbm, gmax_hbm, out_hbm,
            gmax_t, src_v, dst_v, av_s, av_d, ex_v, raw0, raw1,
            dst_sc, scaled_v, acc_sh, si, sr, ss):
    cid = lax.axis_index("c")
    sid = lax.axis_index("s")
    wid = sid * NC + cid
    lane0 = (lax.iota(jnp.int32, 16) == 0).astype(jnp.float32)
    zeros16 = jnp.zeros((16,), jnp.float32)
    izeros16 = jnp.zeros((16,), jnp.int32)
    himask = jnp.full((16,), -65536, jnp.int32)

    pltpu.sync_copy(gmax_hbm, gmax_t)
    gmax = gmax_t[...]

    @pl.loop(0, B)
    def _(r):
        @pl.loop(0, ACCW, step=16)
        def _(c):
            scaled_v[r, pl.ds(c, 16)] = zeros16

    @pl.loop(0, ROWS_PER_TILE // B)
    def _(i):
        pltpu.sync_copy(
            scaled_v, acc_sh.at[pl.ds(sid * ROWS_PER_TILE + i * B, B)])

    pltpu.sync_copy(
        scaled_v.at[pl.ds(0, ROWS_PER_TILE % B)],
        acc_sh.at[pl.ds(sid * ROWS_PER_TILE
                        + (ROWS_PER_TILE // B) * B, ROWS_PER_TILE % B)])

    plsc.subcore_barrier()

    raws = (raw0, raw1)

    def valid(k):
        return wid + k * NW < NBLK

    def idx_start(k, q):
        off = (wid + k * NW) * B
        c0 = pltpu.make_async_copy(ei_hbm.at[0, pl.ds(off, B)],
                                   src_v.at[q], si)
        c1 = pltpu.make_async_copy(ei_hbm.at[1, pl.ds(off, B)],
                                   dst_v.at[q], si)
        c0.start()
        c1.start()
        return (c0, c1)

    def edge_gathers(q, p):
        return (pltpu.make_async_copy(hbi_hbm.at[src_v.at[q]], raws[p], sr),
                pltpu.make_async_copy(asrc_hbm.at[src_v.at[q]],
                                      av_s.at[q], sr),
                pltpu.make_async_copy(adst_hbm.at[dst_v.at[q]],
                                      av_d.at[q], sr))

    idx_start(0, 0)
    pltpu.make_async_copy(ei_hbm.at[0, pl.ds(wid * B, B)],
                          src_v.at[0], si).wait()
    pltpu.make_async_copy(ei_hbm.at[1, pl.ds(wid * B, B)],
                          dst_v.at[0], si).wait()
    for g0 in edge_gathers(0, 0):
        g0.start()
        g0.wait()
    idx_start(1, 1)
    pltpu.make_async_copy(ei_hbm.at[0, pl.ds((wid + NW) * B, B)],
                          src_v.at[1], si).wait()
    pltpu.make_async_copy(ei_hbm.at[1, pl.ds((wid + NW) * B, B)],
                          dst_v.at[1], si).wait()

    def body(k0, k, q):
        kk = k0 + k
        p = q

        @pl.when(valid(kk))
        def _():
            qn = 1 - q

            @pl.when(valid(kk + 1))
            def _():
                for g in edge_gathers(qn, 1 - p):
                    g.start()

            @plsc.parallel_loop(0, B, step=16, unroll=2)
            def _(g):
                a_s = av_s[q, pl.ds(g, 16)]
                a_d = av_d[q, pl.ds(g, 16)]
                m_d = _lrelu(a_d + gmax)
                ex_v[pl.ds(g, 16)] = jnp.exp(_lrelu(a_s + a_d) - m_d)

            raw = raws[p]

            @plsc.parallel_loop(0, B, unroll=4)
            def _(r):
                exr = plsc.load_gather(ex_v, [izeros16 + r])
                for c in range(D // 32):
                    w = raw[r, pl.ds(c * 16, 16)]
                    lo = plsc.bitcast(w << 16, jnp.float32)
                    hi = plsc.bitcast(w & himask, jnp.float32)
                    scaled_v[r, pl.ds(c * 32, 16)] = lo * exr
                    scaled_v[r, pl.ds(c * 32 + 16, 16)] = hi * exr
                scaled_v[r, pl.ds(D, 16)] = exr * lane0

            @pl.loop(0, B, step=16)
            def _(g):
                dst_sc[pl.ds(g, 16)] = dst_v[q, pl.ds(g, 16)]

            sc = pltpu.async_copy(scaled_v, acc_sh.at[dst_sc], ss, add=True)

            @pl.when(valid(kk + 2))
            def _():
                off2 = (wid + (kk + 2) * NW) * B
                c0 = pltpu.make_async_copy(ei_hbm.at[0, pl.ds(off2, B)],
                                           src_v.at[q], si)
                c1 = pltpu.make_async_copy(ei_hbm.at[1, pl.ds(off2, B)],
                                           dst_v.at[q], si)
                c0.start()
                c1.start()
                c0.wait()
                c1.wait()

            sc.wait()

            @pl.when(valid(kk + 1))
            def _():
                for g in edge_gathers(qn, 1 - p):
                    g.wait()

    @pl.loop(0, KMAX + (-KMAX) % 2, step=2)
    def _(k0):
        body(k0, 0, 0)
        body(k0, 1, 1)

    plsc.subcore_barrier()
    pltpu.sync_copy(
        acc_sh.at[pl.ds(sid * ROWS_PER_TILE, ROWS_PER_TILE)],
        out_hbm.at[cid, pl.ds(sid * ROWS_PER_TILE, ROWS_PER_TILE)])


def _gat_layer(x, edge_index, W, att_src, att_dst, bias, act):
    asv = att_src.reshape(1, D).astype(jnp.float32)
    adv = att_dst.reshape(1, D).astype(jnp.float32)
    h, a_src, a_dst, gmax, exl = _pre(x, W, asv, adv)
    hb = h.reshape(N, 4, 2, 16).transpose(0, 1, 3, 2).astype(
        jnp.bfloat16).reshape(N, D // 2, 2)
    hbi = jax.lax.bitcast_convert_type(hb, jnp.int32)
    acc = _sc_gat(edge_index, hbi, a_src.reshape(N), a_dst.reshape(N),
                  jnp.broadcast_to(gmax.reshape(()), (16,)))
    post = _post_elu if act else _post_lin
    return post(acc, h, exl, bias.reshape(1, D))


def kernel(x, edge_index, W1, att_src1, att_dst1, b1, W2, att_src2,
           att_dst2, b2):
    h1 = _gat_layer(x, edge_index, W1, att_src1, att_dst1, b1, act=True)
    return _gat_layer(h1, edge_index, W2, att_src2, att_dst2, b2, act=False)
